# baseline (speedup 1.0000x reference)
import jax
import jax.numpy as jnp
from jax import lax
from jax.experimental import pallas as pl
from jax.experimental.pallas import tpu as pltpu
from jax.experimental.pallas import tpu_sc as plsc

_B, _L = 4096, 50
_NC, _NS, _LANES = 2, 16, 16
_NW = _NC * _NS
_EPW = _B // _NW
_DIMS = (128, 32, 64, 32)
_OFFS = (0, 128, 160, 224)
_DOUT = 256
_FIX_STARTS = (0, 16, 32, 34)


def _body(idx_id, idx_cat, idx_br, idx_sh, w_id, w_cat, w_br, w_sh, out,
          idxv, rid, rcat, rbr, rsh, gsem, wsem):
    wid = lax.axis_index("s") * _NC + lax.axis_index("c")
    e0 = wid * _EPW
    tables = (w_id, w_cat, w_br, w_sh)
    bufs = (rid, rcat, rbr, rsh)

    for t, idx in enumerate((idx_id, idx_cat, idx_br, idx_sh)):
        pltpu.sync_copy(idx.at[pl.ds(e0, _EPW), :], idxv.at[t])

    def gather_descs(e, b):
        return [
            pltpu.make_async_copy(tables[t].at[idxv.at[t, e]],
                                  bufs[t].at[b], gsem)
            for t in range(4)
        ]

    def issue_gathers(e, b):
        for d in gather_descs(e, b):
            d.start()

    def wait_gathers(e, b):
        for d in gather_descs(e, b):
            d.wait()

    def write_descs(e, b):
        return [
            pltpu.make_async_copy(
                bufs[t].at[b],
                out.at[e0 + e, :, pl.ds(_OFFS[t], _DIMS[t])], wsem)
            for t in range(4)
        ]

    def fix_padding(e, b):
        for s in _FIX_STARTS:
            v = idxv[0, e, pl.ds(s, _LANES)]
            m = v == 0
            cnt = jnp.sum(jnp.where(m, 1, 0))

            @pl.when(cnt > 0)
            def _():
                rows = s + lax.iota(jnp.int32, _LANES)
                zeros = jnp.zeros((_LANES,), jnp.float32)

                def fixcol(c, carry):
                    cols = jnp.full((_LANES,), c, jnp.int32)
                    plsc.store_scatter(rid.at[b], [rows, cols], zeros,
                                       mask=m)
                    return carry

                lax.fori_loop(0, _DIMS[0], fixcol, 0)

    issue_gathers(0, 0)

    def elem(e, carry):
        b = lax.rem(e, 2)
        wait_gathers(e, b)

        @pl.when(e >= 1)
        def _():
            for d in write_descs(e - 1, 1 - b):
                d.wait()

        @pl.when(e + 1 < _EPW)
        def _():
            issue_gathers(e + 1, 1 - b)

        fix_padding(e, b)
        for d in write_descs(e, b):
            d.start()
        return carry

    lax.fori_loop(0, _EPW, elem, 0)
    for d in write_descs(_EPW - 1, (_EPW - 1) % 2):
        d.wait()


_gather = pl.kernel(
    _body,
    out_type=jax.ShapeDtypeStruct((_B, _L, _DOUT), jnp.float32),
    mesh=plsc.VectorSubcoreMesh(core_axis_name="c", subcore_axis_name="s",
                                num_cores=_NC, num_subcores=_NS),
    scratch_types=[
        pltpu.VMEM((4, _EPW, _L), jnp.int32),
        pltpu.VMEM((2, _L, _DIMS[0]), jnp.float32),
        pltpu.VMEM((2, _L, _DIMS[1]), jnp.float32),
        pltpu.VMEM((2, _L, _DIMS[2]), jnp.float32),
        pltpu.VMEM((2, _L, _DIMS[3]), jnp.float32),
        pltpu.SemaphoreType.DMA,
        pltpu.SemaphoreType.DMA,
    ],
    compiler_params=pltpu.CompilerParams(use_tc_tiling_on_sc=False,
                                         needs_layout_passes=False),
)


def kernel(attr_id, attr_category, attr_brand, attr_shop,
           W_id, W_category, W_brand, W_shop):
    return _gather(attr_id.astype(jnp.int32), attr_category.astype(jnp.int32),
                   attr_brand.astype(jnp.int32), attr_shop.astype(jnp.int32),
                   W_id, W_category, W_brand, W_shop)


# speedup vs baseline: 1.3725x; 1.3725x over previous
import jax
import jax.numpy as jnp
from jax import lax
from jax.experimental import pallas as pl
from jax.experimental.pallas import tpu as pltpu
from jax.experimental.pallas import tpu_sc as plsc

_B, _L = 4096, 50
_NC, _NS, _LANES = 2, 16, 16
_NW = _NC * _NS
_EPW = _B // _NW
_DOUT = 256
_FIX_STARTS = (0, 16, 32, 34)


def _body(idx_id, idx_cat, idx_br, idx_sh, w_id, cat_p, br_p, sh_p, out,
          idv, icat, ibr, ish,
          rid_0, stg_0, rid_1, stg_1, rid_2, stg_2,
          gsem, csem, wsem):
    wid = lax.axis_index("s") * _NC + lax.axis_index("c")
    e0 = wid * _EPW
    sets = ((rid_0, stg_0), (rid_1, stg_1), (rid_2, stg_2))

    for ref, idx in ((idv, idx_id), (icat, idx_cat), (ibr, idx_br),
                     (ish, idx_sh)):
        pltpu.sync_copy(idx.at[pl.ds(e0, _EPW), :], ref)

    def start_cat(e, s):
        pltpu.async_copy(cat_p.at[icat.at[e]], s[1], csem)

    def wait_cat(e, s):
        pltpu.make_async_copy(cat_p.at[icat.at[e]], s[1], csem).wait()

    def start_others(e, s):
        pltpu.async_copy(w_id.at[idv.at[e]], s[0], gsem)
        pltpu.async_copy(br_p.at[ibr.at[e]], s[1], gsem, add=True)
        pltpu.async_copy(sh_p.at[ish.at[e]], s[1], gsem, add=True)

    def wait_others(e, s):
        pltpu.make_async_copy(w_id.at[idv.at[e]], s[0], gsem).wait()
        pltpu.make_async_copy(br_p.at[ibr.at[e]], s[1], gsem).wait()
        pltpu.make_async_copy(sh_p.at[ish.at[e]], s[1], gsem).wait()

    def write_descs(e, s):
        return [
            pltpu.make_async_copy(s[0], out.at[e0 + e, :, pl.ds(0, 128)],
                                  wsem),
            pltpu.make_async_copy(s[1], out.at[e0 + e, :, pl.ds(128, 128)],
                                  wsem),
        ]

    def fix_padding(e, s):
        for st in _FIX_STARTS:
            v = idv[e, pl.ds(st, _LANES)]
            m = v == 0
            cnt = jnp.sum(jnp.where(m, 1, 0))

            @pl.when(cnt > 0)
            def _():
                rows = st + lax.iota(jnp.int32, _LANES)
                zeros = jnp.zeros((_LANES,), jnp.float32)

                def fixcol(c, carry):
                    cols = jnp.full((_LANES,), c, jnp.int32)
                    plsc.store_scatter(s[0], [rows, cols], zeros, mask=m)
                    return carry

                lax.fori_loop(0, 128, fixcol, 0)

    def process(e, s_cur, s_nxt, s_prv):
        wait_others(e, s_cur)

        @pl.when(e + 1 < _EPW)
        def _():
            wait_cat(e + 1, s_nxt)
            start_others(e + 1, s_nxt)

        @pl.when(e >= 1)
        def _():
            for d in write_descs(e - 1, s_prv):
                d.wait()

        @pl.when(e + 2 < _EPW)
        def _():
            start_cat(e + 2, s_prv)

        fix_padding(e, s_cur)
        for d in write_descs(e, s_cur):
            d.start()

    start_cat(0, sets[0])
    wait_cat(0, sets[0])
    start_others(0, sets[0])
    start_cat(1, sets[1])

    def elem(e, carry):
        r = lax.rem(e, 3)
        for k in range(3):
            @pl.when(r == k)
            def _():
                process(e, sets[k], sets[(k + 1) % 3], sets[(k + 2) % 3])
        return carry

    lax.fori_loop(0, _EPW, elem, 0)
    for d in write_descs(_EPW - 1, sets[(_EPW - 1) % 3]):
        d.wait()


_gather = pl.kernel(
    _body,
    out_type=jax.ShapeDtypeStruct((_B, _L, _DOUT), jnp.float32),
    mesh=plsc.VectorSubcoreMesh(core_axis_name="c", subcore_axis_name="s",
                                num_cores=_NC, num_subcores=_NS),
    scratch_types=[
        pltpu.VMEM((_EPW, _L), jnp.int32),
        pltpu.VMEM((_EPW, _L), jnp.int32),
        pltpu.VMEM((_EPW, _L), jnp.int32),
        pltpu.VMEM((_EPW, _L), jnp.int32),
        pltpu.VMEM((_L, 128), jnp.float32),
        pltpu.VMEM((_L, 128), jnp.float32),
        pltpu.VMEM((_L, 128), jnp.float32),
        pltpu.VMEM((_L, 128), jnp.float32),
        pltpu.VMEM((_L, 128), jnp.float32),
        pltpu.VMEM((_L, 128), jnp.float32),
        pltpu.SemaphoreType.DMA,
        pltpu.SemaphoreType.DMA,
        pltpu.SemaphoreType.DMA,
    ],
    compiler_params=pltpu.CompilerParams(needs_layout_passes=False),
)


def kernel(attr_id, attr_category, attr_brand, attr_shop,
           W_id, W_category, W_brand, W_shop):
    cat_p = jnp.pad(W_category, ((0, 0), (0, 96)))
    br_p = jnp.pad(W_brand, ((0, 0), (32, 32)))
    sh_p = jnp.pad(W_shop, ((0, 0), (96, 0)))
    return _gather(attr_id.astype(jnp.int32), attr_category.astype(jnp.int32),
                   attr_brand.astype(jnp.int32), attr_shop.astype(jnp.int32),
                   W_id, cat_p, br_p, sh_p)
